# baseline (speedup 1.0000x reference)
import jax
import jax.numpy as jnp
from jax import lax
from jax.experimental import pallas as pl
from jax.experimental.pallas import tpu as pltpu
from jax.experimental.pallas import tpu_sc as plsc

_N = 100000
_E = 3200000
_Q = 9
_DT = 0.1

_NC = 2
_NS = 16
_NW = _NC * _NS
_SUB = 128
_CHUNK = 1024
_NSUB = _CHUNK // _SUB
_EPW = 102400
_PADW = _EPW - _E // _NW
_NCHUNKS = _EPW // _CHUNK
_NDUMP = 96
_AW = 16
_RPT = 6256
_NPAD = _RPT * _NS


def _sc_body(fpad, src2, dst2, w1, zrows, out, si, di, wv, r16, acc,
             sem_g, sem_s):
    c = lax.axis_index("c")
    s = lax.axis_index("s")

    pltpu.sync_copy(zrows, r16)
    nfull = _RPT // _CHUNK
    rem = _RPT - nfull * _CHUNK
    for t in range(nfull):
        pltpu.sync_copy(r16, acc.at[pl.ds(s * _RPT + t * _CHUNK, _CHUNK)])
    if rem:
        pltpu.sync_copy(r16.at[pl.ds(0, rem)],
                        acc.at[pl.ds(s * _RPT + nfull * _CHUNK, rem)])
    plsc.subcore_barrier()

    wid = c * _NS + s
    base_e = wid * _EPW
    it = lax.broadcasted_iota(jnp.int32, (16,), 0)
    ones16 = jnp.ones((16,), jnp.float32)
    col10 = jnp.full((16,), _Q + 1, jnp.int32)

    def chunk_body(k, carry):
        row0 = k * _NSUB
        pltpu.sync_copy(src2.at[wid, pl.ds(row0, _NSUB)], si)
        pltpu.sync_copy(dst2.at[wid, pl.ds(row0, _NSUB)], di)
        pltpu.sync_copy(w1.at[pl.ds(base_e + k * _CHUNK, _CHUNK)], wv)

        cps = [
            pltpu.async_copy(fpad.at[si.at[j]],
                             r16.at[pl.ds(j * _SUB, _SUB)], sem_g)
            for j in range(_NSUB)
        ]
        for cp in cps:
            cp.wait()

        def grp_body(g, carry2):
            ridx = it + g * 16
            wg = wv[pl.ds(g * 16, 16)]
            for q in range(_Q + 1):
                col = jnp.full((16,), q, jnp.int32)
                v = plsc.load_gather(r16, [ridx, col])
                plsc.store_scatter(r16, [ridx, col], v * wg)
            plsc.store_scatter(r16, [ridx, col10], ones16)
            return carry2

        lax.fori_loop(0, _CHUNK // 16, grp_body, 0)

        scps = [
            pltpu.async_copy(r16.at[pl.ds(j * _SUB, _SUB)],
                             acc.at[di.at[j]], sem_s, add=True)
            for j in range(_NSUB)
        ]
        for cp in scps:
            cp.wait()
        return carry

    lax.fori_loop(0, _NCHUNKS, chunk_body, 0)

    plsc.subcore_barrier()
    for t in range(nfull):
        r0 = s * _RPT + t * _CHUNK
        pltpu.sync_copy(acc.at[pl.ds(r0, _CHUNK)], r16)
        pltpu.sync_copy(r16, out.at[c, pl.ds(r0, _CHUNK)])
    if rem:
        r0 = s * _RPT + nfull * _CHUNK
        pltpu.sync_copy(acc.at[pl.ds(r0, rem)], r16.at[pl.ds(0, rem)])
        pltpu.sync_copy(r16.at[pl.ds(0, rem)], out.at[c, pl.ds(r0, rem)])


_sc_call = pl.kernel(
    _sc_body,
    out_type=jax.ShapeDtypeStruct((_NC, _NPAD, _AW), jnp.float32),
    mesh=plsc.VectorSubcoreMesh(core_axis_name="c", subcore_axis_name="s"),
    compiler_params=pltpu.CompilerParams(use_tc_tiling_on_sc=False,
                                         needs_layout_passes=False),
    scratch_types=[
        pltpu.VMEM((_NSUB, _SUB), jnp.int32),
        pltpu.VMEM((_NSUB, _SUB), jnp.int32),
        pltpu.VMEM((_CHUNK,), jnp.float32),
        pltpu.VMEM((_CHUNK, 16), jnp.float32),
        pltpu.VMEM_SHARED((_NPAD, _AW), jnp.float32),
        pltpu.SemaphoreType.DMA,
        pltpu.SemaphoreType.DMA,
    ],
)

_BN = 2000


def _tc_body(f_ref, a_ref, coll_ref, srct_ref, xi_ref, o_ref):
    a = a_ref[0] + a_ref[1]
    f = f_ref[...]
    ssum = a[:, :_Q]
    wsum = a[:, _Q:_Q + 1]
    deg = jnp.maximum(a[:, _Q + 1:_Q + 2], 1.0)
    transport = (ssum - f * wsum) / deg * xi_ref[...]
    o_ref[...] = f - _DT * (transport - coll_ref[...] + srct_ref[...])


_tc_call = pl.pallas_call(
    _tc_body,
    out_shape=jax.ShapeDtypeStruct((_N, _Q), jnp.float32),
    grid=(_N // _BN,),
    in_specs=[
        pl.BlockSpec((_BN, _Q), lambda i: (i, 0)),
        pl.BlockSpec((_NC, _BN, _AW), lambda i: (0, i, 0)),
        pl.BlockSpec((_BN, _Q), lambda i: (i, 0)),
        pl.BlockSpec((_BN, _Q), lambda i: (i, 0)),
        pl.BlockSpec((1, _Q), lambda i: (0, 0)),
    ],
    out_specs=pl.BlockSpec((_BN, _Q), lambda i: (i, 0)),
)


def kernel(f_distribution, edge_index, edge_weight, collision_term,
           source_term, xi_velocities):
    epw = _E // _NW
    src_pad = jnp.zeros((_NW, _PADW), jnp.int32)
    dst_pad = jnp.broadcast_to(
        _N + (jnp.arange(_PADW, dtype=jnp.int32) % _NDUMP), (_NW, _PADW))
    src2 = jnp.concatenate(
        [edge_index[0].astype(jnp.int32).reshape(_NW, epw), src_pad],
        axis=1).reshape(_NW, _EPW // _SUB, _SUB)
    dst2 = jnp.concatenate(
        [edge_index[1].astype(jnp.int32).reshape(_NW, epw), dst_pad],
        axis=1).reshape(_NW, _EPW // _SUB, _SUB)
    w1 = jnp.concatenate(
        [edge_weight.reshape(_NW, epw),
         jnp.zeros((_NW, _PADW), jnp.float32)], axis=1).reshape(-1)
    fpad = jnp.concatenate(
        [f_distribution, jnp.ones((_N, 1), jnp.float32),
         jnp.zeros((_N, 15 - _Q), jnp.float32)], axis=1)
    zrows = jnp.zeros((_CHUNK, _AW), jnp.float32)
    a = _sc_call(fpad, src2, dst2, w1, zrows)
    xi2 = xi_velocities.reshape(1, _Q)
    return _tc_call(f_distribution, a, collision_term, source_term, xi2)


# speedup vs baseline: 1.4284x; 1.4284x over previous
import jax
import jax.numpy as jnp
from jax import lax
from jax.experimental import pallas as pl
from jax.experimental.pallas import tpu as pltpu
from jax.experimental.pallas import tpu_sc as plsc

_N = 100000
_E = 3200000
_Q = 9
_DT = 0.1

_NC = 2
_NS = 16
_NW = _NC * _NS
_SUB = 128
_CHUNK = 1024
_NSUB = _CHUNK // _SUB
_NROWS = _E // _SUB
_RPW = _NROWS // _NW
_NCK = (_RPW - 5) // _NSUB
_TROWS = _RPW - _NCK * _NSUB
_AW = 16
_RPT = 6256
_NPAD = _RPT * _NS


def _sc_body(fpad, src2, dst2, w1, zrows, out, si, di, wv, r16, acc,
             sem_g, sem_s):
    c = lax.axis_index("c")
    s = lax.axis_index("s")

    pltpu.sync_copy(zrows, r16)
    nfull = _RPT // _CHUNK
    rem = _RPT - nfull * _CHUNK
    for t in range(nfull):
        pltpu.sync_copy(r16, acc.at[pl.ds(s * _RPT + t * _CHUNK, _CHUNK)])
    if rem:
        pltpu.sync_copy(r16.at[pl.ds(0, rem)],
                        acc.at[pl.ds(s * _RPT + nfull * _CHUNK, rem)])
    plsc.subcore_barrier()

    wid = c * _NS + s
    base_row = wid * _RPW
    base_e = base_row * _SUB
    nck = _NCK + jnp.where(wid == _NW - 1, 1, 0)
    it = lax.broadcasted_iota(jnp.int32, (16,), 0)
    ones16 = jnp.ones((16,), jnp.float32)
    col10 = jnp.full((16,), _Q + 1, jnp.int32)

    def grp_body(g, carry2):
        ridx = it + g * 16
        wg = wv[pl.ds(g * 16, 16)]
        for q in range(_Q + 1):
            col = jnp.full((16,), q, jnp.int32)
            v = plsc.load_gather(r16, [ridx, col])
            plsc.store_scatter(r16, [ridx, col], v * wg)
        plsc.store_scatter(r16, [ridx, col10], ones16)
        return carry2

    def chunk_body(k, carry):
        row0 = base_row + k * _NSUB
        pltpu.sync_copy(src2.at[pl.ds(row0, _NSUB)], si)
        pltpu.sync_copy(dst2.at[pl.ds(row0, _NSUB)], di)
        pltpu.sync_copy(w1.at[pl.ds(base_e + k * _CHUNK, _CHUNK)], wv)

        cps = [
            pltpu.async_copy(fpad.at[si.at[j]],
                             r16.at[pl.ds(j * _SUB, _SUB)], sem_g)
            for j in range(_NSUB)
        ]
        for cp in cps:
            cp.wait()

        lax.fori_loop(0, _CHUNK // 16, grp_body, 0)

        scps = [
            pltpu.async_copy(r16.at[pl.ds(j * _SUB, _SUB)],
                             acc.at[di.at[j]], sem_s, add=True)
            for j in range(_NSUB)
        ]
        for cp in scps:
            cp.wait()
        return carry

    lax.fori_loop(0, nck, chunk_body, 0)

    trow = base_row + nck * _NSUB
    te0 = base_e + nck * _CHUNK
    pltpu.sync_copy(src2.at[pl.ds(trow, _TROWS)], si.at[pl.ds(0, _TROWS)])
    pltpu.sync_copy(dst2.at[pl.ds(trow, _TROWS)], di.at[pl.ds(0, _TROWS)])
    pltpu.sync_copy(w1.at[pl.ds(te0, _TROWS * _SUB)],
                    wv.at[pl.ds(0, _TROWS * _SUB)])
    cps = [
        pltpu.async_copy(fpad.at[si.at[j]],
                         r16.at[pl.ds(j * _SUB, _SUB)], sem_g)
        for j in range(_TROWS)
    ]
    for cp in cps:
        cp.wait()
    lax.fori_loop(0, _TROWS * _SUB // 16, grp_body, 0)
    scps = [
        pltpu.async_copy(r16.at[pl.ds(j * _SUB, _SUB)],
                         acc.at[di.at[j]], sem_s, add=True)
        for j in range(_TROWS)
    ]
    for cp in scps:
        cp.wait()

    plsc.subcore_barrier()
    for t in range(nfull):
        r0 = s * _RPT + t * _CHUNK
        pltpu.sync_copy(acc.at[pl.ds(r0, _CHUNK)], r16)
        pltpu.sync_copy(r16, out.at[c, pl.ds(r0, _CHUNK)])
    if rem:
        r0 = s * _RPT + nfull * _CHUNK
        pltpu.sync_copy(acc.at[pl.ds(r0, rem)], r16.at[pl.ds(0, rem)])
        pltpu.sync_copy(r16.at[pl.ds(0, rem)], out.at[c, pl.ds(r0, rem)])


_sc_call = pl.kernel(
    _sc_body,
    out_type=jax.ShapeDtypeStruct((_NC, _NPAD, _AW), jnp.float32),
    mesh=plsc.VectorSubcoreMesh(core_axis_name="c", subcore_axis_name="s"),
    compiler_params=pltpu.CompilerParams(use_tc_tiling_on_sc=False,
                                         needs_layout_passes=False),
    scratch_types=[
        pltpu.VMEM((_NSUB, _SUB), jnp.int32),
        pltpu.VMEM((_NSUB, _SUB), jnp.int32),
        pltpu.VMEM((_CHUNK,), jnp.float32),
        pltpu.VMEM((_CHUNK, 16), jnp.float32),
        pltpu.VMEM_SHARED((_NPAD, _AW), jnp.float32),
        pltpu.SemaphoreType.DMA,
        pltpu.SemaphoreType.DMA,
    ],
)

_BN = 2000


def _tc_body(f_ref, a_ref, coll_ref, srct_ref, xi_ref, o_ref):
    a = a_ref[0] + a_ref[1]
    f = f_ref[...]
    ssum = a[:, :_Q]
    wsum = a[:, _Q:_Q + 1]
    deg = jnp.maximum(a[:, _Q + 1:_Q + 2], 1.0)
    transport = (ssum - f * wsum) / deg * xi_ref[...]
    o_ref[...] = f - _DT * (transport - coll_ref[...] + srct_ref[...])


_tc_call = pl.pallas_call(
    _tc_body,
    out_shape=jax.ShapeDtypeStruct((_N, _Q), jnp.float32),
    grid=(_N // _BN,),
    in_specs=[
        pl.BlockSpec((_BN, _Q), lambda i: (i, 0)),
        pl.BlockSpec((_NC, _BN, _AW), lambda i: (0, i, 0)),
        pl.BlockSpec((_BN, _Q), lambda i: (i, 0)),
        pl.BlockSpec((_BN, _Q), lambda i: (i, 0)),
        pl.BlockSpec((1, _Q), lambda i: (0, 0)),
    ],
    out_specs=pl.BlockSpec((_BN, _Q), lambda i: (i, 0)),
)


def kernel(f_distribution, edge_index, edge_weight, collision_term,
           source_term, xi_velocities):
    src2 = edge_index[0].astype(jnp.int32).reshape(_NROWS, _SUB)
    dst2 = edge_index[1].astype(jnp.int32).reshape(_NROWS, _SUB)
    w1 = edge_weight
    fpad = jnp.concatenate(
        [f_distribution, jnp.ones((_N, 1), jnp.float32),
         jnp.zeros((_N, 15 - _Q), jnp.float32)], axis=1)
    zrows = jnp.zeros((_CHUNK, _AW), jnp.float32)
    a = _sc_call(fpad, src2, dst2, w1, zrows)
    xi2 = xi_velocities.reshape(1, _Q)
    return _tc_call(f_distribution, a, collision_term, source_term, xi2)


# speedup vs baseline: 1.5385x; 1.0771x over previous
import jax
import jax.numpy as jnp
from jax import lax
from jax.experimental import pallas as pl
from jax.experimental.pallas import tpu as pltpu
from jax.experimental.pallas import tpu_sc as plsc

_N = 100000
_E = 3200000
_Q = 9
_DT = 0.1

_NC = 2
_NS = 16
_NW = _NC * _NS
_SUB = 128
_NROWS = _E // _SUB
_QR = 4
_SLOT = _QR * _SUB
_PAIRS = 97
_MRPW = _PAIRS * 2 * _QR
_MROWS = _MRPW * _NW
_REMW = (_NROWS - _MROWS) // (2 * _QR)
_AW = 16
_RPT = 6256
_NPAD = _RPT * _NS


def _sc_body(fpad, src2, dst2, w1, zrows, out,
             si0, di0, wv0, rb0, si1, di1, wv1, rb1, acc,
             sem_g0, sem_g1, sem_s0, sem_s1):
    c = lax.axis_index("c")
    s = lax.axis_index("s")

    pltpu.sync_copy(zrows, rb0)
    nfull = _RPT // _SLOT
    rem = _RPT - nfull * _SLOT
    for t in range(nfull):
        pltpu.sync_copy(rb0, acc.at[pl.ds(s * _RPT + t * _SLOT, _SLOT)])
    if rem:
        pltpu.sync_copy(rb0.at[pl.ds(0, rem)],
                        acc.at[pl.ds(s * _RPT + nfull * _SLOT, rem)])
    plsc.subcore_barrier()

    wid = c * _NS + s
    base_row = wid * _MRPW
    npairs = _PAIRS + jnp.where(wid < _REMW, 1, 0)
    it = lax.broadcasted_iota(jnp.int32, (16,), 0)
    ones16 = jnp.ones((16,), jnp.float32)
    col10 = jnp.full((16,), _Q + 1, jnp.int32)
    slots = ((si0, di0, wv0, rb0, sem_g0, sem_s0),
             (si1, di1, wv1, rb1, sem_g1, sem_s1))

    def make_grp(rb, wv):
        def grp_body(g, carry2):
            ridx = it + g * 16
            wg = wv[pl.ds(g * 16, 16)]
            for q in range(_Q + 1):
                col = jnp.full((16,), q, jnp.int32)
                v = plsc.load_gather(rb, [ridx, col])
                plsc.store_scatter(rb, [ridx, col], v * wg)
            plsc.store_scatter(rb, [ridx, col10], ones16)
            return carry2
        return grp_body

    def _drain(rb, sem):
        pltpu.make_async_copy(fpad.at[pl.ds(0, _SLOT)], rb, sem).wait()

    def pair_body(p, carry):
        for b, (si, di, wv, rb, sg, ss) in enumerate(slots):
            row = jnp.where(p < _PAIRS, base_row + p * 2 * _QR,
                            _MROWS + 2 * _QR * wid) + b * _QR

            @pl.when(p > 0)
            def _():
                _drain(rb, ss)

            pltpu.sync_copy(src2.at[pl.ds(row, _QR)], si)
            pltpu.sync_copy(dst2.at[pl.ds(row, _QR)], di)
            pltpu.sync_copy(w1.at[pl.ds(row * _SUB, _SLOT)], wv)
            for j in range(_QR):
                pltpu.async_copy(fpad.at[si.at[j]],
                                 rb.at[pl.ds(j * _SUB, _SUB)], sg)
        for b, (si, di, wv, rb, sg, ss) in enumerate(slots):
            _drain(rb, sg)
            lax.fori_loop(0, _SLOT // 16, make_grp(rb, wv), 0)
            for j in range(_QR):
                pltpu.async_copy(rb.at[pl.ds(j * _SUB, _SUB)],
                                 acc.at[di.at[j]], ss, add=True)
        return carry

    lax.fori_loop(0, npairs, pair_body, 0)
    for b, (si, di, wv, rb, sg, ss) in enumerate(slots):
        _drain(rb, ss)

    plsc.subcore_barrier()
    for t in range(nfull):
        r0 = s * _RPT + t * _SLOT
        pltpu.sync_copy(acc.at[pl.ds(r0, _SLOT)], rb0)
        pltpu.sync_copy(rb0, out.at[c, pl.ds(r0, _SLOT)])
    if rem:
        r0 = s * _RPT + nfull * _SLOT
        pltpu.sync_copy(acc.at[pl.ds(r0, rem)], rb0.at[pl.ds(0, rem)])
        pltpu.sync_copy(rb0.at[pl.ds(0, rem)], out.at[c, pl.ds(r0, rem)])


_sc_call = pl.kernel(
    _sc_body,
    out_type=jax.ShapeDtypeStruct((_NC, _NPAD, _AW), jnp.float32),
    mesh=plsc.VectorSubcoreMesh(core_axis_name="c", subcore_axis_name="s"),
    compiler_params=pltpu.CompilerParams(use_tc_tiling_on_sc=False,
                                         needs_layout_passes=False),
    scratch_types=[
        pltpu.VMEM((_QR, _SUB), jnp.int32),
        pltpu.VMEM((_QR, _SUB), jnp.int32),
        pltpu.VMEM((_SLOT,), jnp.float32),
        pltpu.VMEM((_SLOT, 16), jnp.float32),
        pltpu.VMEM((_QR, _SUB), jnp.int32),
        pltpu.VMEM((_QR, _SUB), jnp.int32),
        pltpu.VMEM((_SLOT,), jnp.float32),
        pltpu.VMEM((_SLOT, 16), jnp.float32),
        pltpu.VMEM_SHARED((_NPAD, _AW), jnp.float32),
        pltpu.SemaphoreType.DMA,
        pltpu.SemaphoreType.DMA,
        pltpu.SemaphoreType.DMA,
        pltpu.SemaphoreType.DMA,
    ],
)

_BN = 2000


def _tc_body(f_ref, a_ref, coll_ref, srct_ref, xi_ref, o_ref):
    a = a_ref[0] + a_ref[1]
    f = f_ref[...]
    ssum = a[:, :_Q]
    wsum = a[:, _Q:_Q + 1]
    deg = jnp.maximum(a[:, _Q + 1:_Q + 2], 1.0)
    transport = (ssum - f * wsum) / deg * xi_ref[...]
    o_ref[...] = f - _DT * (transport - coll_ref[...] + srct_ref[...])


_tc_call = pl.pallas_call(
    _tc_body,
    out_shape=jax.ShapeDtypeStruct((_N, _Q), jnp.float32),
    grid=(_N // _BN,),
    in_specs=[
        pl.BlockSpec((_BN, _Q), lambda i: (i, 0)),
        pl.BlockSpec((_NC, _BN, _AW), lambda i: (0, i, 0)),
        pl.BlockSpec((_BN, _Q), lambda i: (i, 0)),
        pl.BlockSpec((_BN, _Q), lambda i: (i, 0)),
        pl.BlockSpec((1, _Q), lambda i: (0, 0)),
    ],
    out_specs=pl.BlockSpec((_BN, _Q), lambda i: (i, 0)),
)


def kernel(f_distribution, edge_index, edge_weight, collision_term,
           source_term, xi_velocities):
    src2 = edge_index[0].astype(jnp.int32).reshape(_NROWS, _SUB)
    dst2 = edge_index[1].astype(jnp.int32).reshape(_NROWS, _SUB)
    w1 = edge_weight
    fpad = jnp.concatenate(
        [f_distribution, jnp.ones((_N, 1), jnp.float32),
         jnp.zeros((_N, 15 - _Q), jnp.float32)], axis=1)
    zrows = jnp.zeros((_SLOT, _AW), jnp.float32)
    a = _sc_call(fpad, src2, dst2, w1, zrows)
    xi2 = xi_velocities.reshape(1, _Q)
    return _tc_call(f_distribution, a, collision_term, source_term, xi2)


# speedup vs baseline: 1.5525x; 1.0091x over previous
import jax
import jax.numpy as jnp
from jax import lax
from jax.experimental import pallas as pl
from jax.experimental.pallas import tpu as pltpu
from jax.experimental.pallas import tpu_sc as plsc

_N = 100000
_E = 3200000
_Q = 9
_DT = 0.1

_NC = 2
_NS = 16
_NW = _NC * _NS
_SUB = 128
_NROWS = _E // _SUB
_QR = 4
_SLOT = _QR * _SUB
_PAIRS = 97
_MRPW = _PAIRS * 2 * _QR
_MROWS = _MRPW * _NW
_REMW = (_NROWS - _MROWS) // (2 * _QR)
_AW = 16
_RPT = 6256
_NPAD = _RPT * _NS


def _sc_body(fpad, src2, w1, zrows, out,
             si0, di0, wv0, rb0, si1, di1, wv1, rb1, acc,
             sem_g0, sem_g1, sem_s0, sem_s1):
    c = lax.axis_index("c")
    s = lax.axis_index("s")

    pltpu.sync_copy(zrows, rb0)
    nfull = _RPT // _SLOT
    rem = _RPT - nfull * _SLOT
    for t in range(nfull):
        pltpu.sync_copy(rb0, acc.at[pl.ds(s * _RPT + t * _SLOT, _SLOT)])
    if rem:
        pltpu.sync_copy(rb0.at[pl.ds(0, rem)],
                        acc.at[pl.ds(s * _RPT + nfull * _SLOT, rem)])
    plsc.subcore_barrier()

    wid = c * _NS + s
    base_row = wid * _MRPW
    npairs = _PAIRS + jnp.where(wid < _REMW, 1, 0)
    it = lax.broadcasted_iota(jnp.int32, (16,), 0)
    ones16 = jnp.ones((16,), jnp.float32)
    col10 = jnp.full((16,), _Q + 1, jnp.int32)
    slots = ((si0, di0, wv0, rb0, sem_g0, sem_s0),
             (si1, di1, wv1, rb1, sem_g1, sem_s1))

    def make_grp(rb, wv):
        def grp_body(g, carry2):
            ridx = it + g * 16
            wg = wv[pl.ds(g * 16, 16)]
            for q in range(_Q + 1):
                col = jnp.full((16,), q, jnp.int32)
                v = plsc.load_gather(rb, [ridx, col])
                plsc.store_scatter(rb, [ridx, col], v * wg)
            plsc.store_scatter(rb, [ridx, col10], ones16)
            return carry2
        return grp_body

    def _drain(rb, sem):
        pltpu.make_async_copy(fpad.at[pl.ds(0, _SLOT)], rb, sem).wait()

    def pair_body(p, carry):
        for b, (si, di, wv, rb, sg, ss) in enumerate(slots):
            row = jnp.where(p < _PAIRS, base_row + p * 2 * _QR,
                            _MROWS + 2 * _QR * wid) + b * _QR

            @pl.when(p > 0)
            def _():
                _drain(rb, ss)

            e0 = row * _SUB
            pltpu.sync_copy(src2.at[0, pl.ds(e0, _SLOT)], si)
            pltpu.sync_copy(src2.at[1, pl.ds(e0, _SLOT)], di)
            pltpu.sync_copy(w1.at[pl.ds(e0, _SLOT)], wv)
            for j in range(_QR):
                pltpu.async_copy(fpad.at[si.at[pl.ds(j * _SUB, _SUB)]],
                                 rb.at[pl.ds(j * _SUB, _SUB)], sg)
        for b, (si, di, wv, rb, sg, ss) in enumerate(slots):
            _drain(rb, sg)
            lax.fori_loop(0, _SLOT // 16, make_grp(rb, wv), 0)
            for j in range(_QR):
                pltpu.async_copy(rb.at[pl.ds(j * _SUB, _SUB)],
                                 acc.at[di.at[pl.ds(j * _SUB, _SUB)]],
                                 ss, add=True)
        return carry

    lax.fori_loop(0, npairs, pair_body, 0)
    for b, (si, di, wv, rb, sg, ss) in enumerate(slots):
        _drain(rb, ss)

    plsc.subcore_barrier()
    for t in range(nfull):
        r0 = s * _RPT + t * _SLOT
        pltpu.sync_copy(acc.at[pl.ds(r0, _SLOT)], rb0)
        pltpu.sync_copy(rb0, out.at[c, pl.ds(r0, _SLOT)])
    if rem:
        r0 = s * _RPT + nfull * _SLOT
        pltpu.sync_copy(acc.at[pl.ds(r0, rem)], rb0.at[pl.ds(0, rem)])
        pltpu.sync_copy(rb0.at[pl.ds(0, rem)], out.at[c, pl.ds(r0, rem)])


_sc_call = pl.kernel(
    _sc_body,
    out_type=jax.ShapeDtypeStruct((_NC, _NPAD, _AW), jnp.float32),
    mesh=plsc.VectorSubcoreMesh(core_axis_name="c", subcore_axis_name="s"),
    compiler_params=pltpu.CompilerParams(use_tc_tiling_on_sc=False,
                                         needs_layout_passes=False),
    scratch_types=[
        pltpu.VMEM((_SLOT,), jnp.int32),
        pltpu.VMEM((_SLOT,), jnp.int32),
        pltpu.VMEM((_SLOT,), jnp.float32),
        pltpu.VMEM((_SLOT, 16), jnp.float32),
        pltpu.VMEM((_SLOT,), jnp.int32),
        pltpu.VMEM((_SLOT,), jnp.int32),
        pltpu.VMEM((_SLOT,), jnp.float32),
        pltpu.VMEM((_SLOT, 16), jnp.float32),
        pltpu.VMEM_SHARED((_NPAD, _AW), jnp.float32),
        pltpu.SemaphoreType.DMA,
        pltpu.SemaphoreType.DMA,
        pltpu.SemaphoreType.DMA,
        pltpu.SemaphoreType.DMA,
    ],
)

_BN = 2000


def _tc_body(f_ref, a_ref, coll_ref, srct_ref, xi_ref, o_ref):
    a = a_ref[0] + a_ref[1]
    f = f_ref[...]
    ssum = a[:, :_Q]
    wsum = a[:, _Q:_Q + 1]
    deg = jnp.maximum(a[:, _Q + 1:_Q + 2], 1.0)
    transport = (ssum - f * wsum) / deg * xi_ref[...]
    o_ref[...] = f - _DT * (transport - coll_ref[...] + srct_ref[...])


_tc_call = pl.pallas_call(
    _tc_body,
    out_shape=jax.ShapeDtypeStruct((_N, _Q), jnp.float32),
    grid=(_N // _BN,),
    in_specs=[
        pl.BlockSpec((_BN, _Q), lambda i: (i, 0)),
        pl.BlockSpec((_NC, _BN, _AW), lambda i: (0, i, 0)),
        pl.BlockSpec((_BN, _Q), lambda i: (i, 0)),
        pl.BlockSpec((_BN, _Q), lambda i: (i, 0)),
        pl.BlockSpec((1, _Q), lambda i: (0, 0)),
    ],
    out_specs=pl.BlockSpec((_BN, _Q), lambda i: (i, 0)),
)


def kernel(f_distribution, edge_index, edge_weight, collision_term,
           source_term, xi_velocities):
    src2 = edge_index.astype(jnp.int32)
    w1 = edge_weight
    fpad = jnp.concatenate(
        [f_distribution, jnp.ones((_N, 1), jnp.float32),
         jnp.zeros((_N, 15 - _Q), jnp.float32)], axis=1)
    zrows = jnp.zeros((_SLOT, _AW), jnp.float32)
    a = _sc_call(fpad, src2, w1, zrows)
    xi2 = xi_velocities.reshape(1, _Q)
    return _tc_call(f_distribution, a, collision_term, source_term, xi2)


# speedup vs baseline: 1.5561x; 1.0023x over previous
import jax
import jax.numpy as jnp
from jax import lax
from jax.experimental import pallas as pl
from jax.experimental.pallas import tpu as pltpu
from jax.experimental.pallas import tpu_sc as plsc

_N = 100000
_E = 3200000
_Q = 9
_DT = 0.1

_NC = 2
_NS = 16
_NW = _NC * _NS
_SUB = 128
_NROWS = _E // _SUB
_QR = 4
_SLOT = _QR * _SUB
_PAIRS = 97
_MRPW = _PAIRS * 2 * _QR
_MROWS = _MRPW * _NW
_REMW = (_NROWS - _MROWS) // (2 * _QR)
_AW = 16
_RPT = 6256
_NPAD = _RPT * _NS


def _sc_body(fpad, src2, w1, zrows, out,
             si0, di0, wv0, rb0, si1, di1, wv1, rb1, acc,
             sem_g0, sem_g1, sem_s0, sem_s1):
    c = lax.axis_index("c")
    s = lax.axis_index("s")

    pltpu.sync_copy(zrows, rb0)
    nfull = _RPT // _SLOT
    rem = _RPT - nfull * _SLOT
    for t in range(nfull):
        pltpu.sync_copy(rb0, acc.at[pl.ds(s * _RPT + t * _SLOT, _SLOT)])
    if rem:
        pltpu.sync_copy(rb0.at[pl.ds(0, rem)],
                        acc.at[pl.ds(s * _RPT + nfull * _SLOT, rem)])
    plsc.subcore_barrier()

    wid = c * _NS + s
    base_row = wid * _MRPW
    npairs = _PAIRS + jnp.where(wid < _REMW, 1, 0)
    it = lax.broadcasted_iota(jnp.int32, (16,), 0)
    ones16 = jnp.ones((16,), jnp.float32)
    col10 = jnp.full((16,), _Q + 1, jnp.int32)
    slots = ((si0, di0, wv0, rb0, sem_g0, sem_s0),
             (si1, di1, wv1, rb1, sem_g1, sem_s1))

    def make_grp(rb, wv):
        def grp_body(g, carry2):
            ridx = it + g * 16
            wg = wv[pl.ds(g * 16, 16)]
            for q in range(_Q + 1):
                col = jnp.full((16,), q, jnp.int32)
                v = plsc.load_gather(rb, [ridx, col])
                plsc.store_scatter(rb, [ridx, col], v * wg)
            plsc.store_scatter(rb, [ridx, col10], ones16)
            return carry2
        return grp_body

    def _drain(rb, sem):
        pltpu.make_async_copy(fpad.at[pl.ds(0, _SLOT)], rb, sem).wait()

    def pair_body(p, carry):
        for b, (si, di, wv, rb, sg, ss) in enumerate(slots):
            row = jnp.where(p < _PAIRS, base_row + p * 2 * _QR,
                            _MROWS + 2 * _QR * wid) + b * _QR

            @pl.when(p > 0)
            def _():
                _drain(rb, ss)

            e0 = row * _SUB
            pltpu.sync_copy(src2.at[0, pl.ds(e0, _SLOT)], si)
            pltpu.sync_copy(src2.at[1, pl.ds(e0, _SLOT)], di)
            pltpu.sync_copy(w1.at[pl.ds(e0, _SLOT)], wv)
            for j in range(_QR):
                pltpu.async_copy(fpad.at[si.at[pl.ds(j * _SUB, _SUB)]],
                                 rb.at[pl.ds(j * _SUB, _SUB)], sg)
        for b, (si, di, wv, rb, sg, ss) in enumerate(slots):
            _drain(rb, sg)
            lax.fori_loop(0, _SLOT // 16, make_grp(rb, wv), 0)
            for j in range(_QR):
                pltpu.async_copy(rb.at[pl.ds(j * _SUB, _SUB)],
                                 acc.at[di.at[pl.ds(j * _SUB, _SUB)]],
                                 ss, add=True)
        return carry

    lax.fori_loop(0, npairs, pair_body, 0)
    for b, (si, di, wv, rb, sg, ss) in enumerate(slots):
        _drain(rb, ss)

    plsc.subcore_barrier()
    for t in range(nfull):
        r0 = s * _RPT + t * _SLOT
        pltpu.sync_copy(acc.at[pl.ds(r0, _SLOT)], rb0)
        pltpu.sync_copy(rb0, out.at[c, pl.ds(r0, _SLOT)])
    if rem:
        r0 = s * _RPT + nfull * _SLOT
        pltpu.sync_copy(acc.at[pl.ds(r0, rem)], rb0.at[pl.ds(0, rem)])
        pltpu.sync_copy(rb0.at[pl.ds(0, rem)], out.at[c, pl.ds(r0, rem)])


_sc_call = pl.kernel(
    _sc_body,
    out_type=jax.ShapeDtypeStruct((_NC, _NPAD, _AW), jnp.float32),
    mesh=plsc.VectorSubcoreMesh(core_axis_name="c", subcore_axis_name="s"),
    compiler_params=pltpu.CompilerParams(use_tc_tiling_on_sc=False,
                                         needs_layout_passes=False),
    scratch_types=[
        pltpu.VMEM((_SLOT,), jnp.int32),
        pltpu.VMEM((_SLOT,), jnp.int32),
        pltpu.VMEM((_SLOT,), jnp.float32),
        pltpu.VMEM((_SLOT, 16), jnp.float32),
        pltpu.VMEM((_SLOT,), jnp.int32),
        pltpu.VMEM((_SLOT,), jnp.int32),
        pltpu.VMEM((_SLOT,), jnp.float32),
        pltpu.VMEM((_SLOT, 16), jnp.float32),
        pltpu.VMEM_SHARED((_NPAD, _AW), jnp.float32),
        pltpu.SemaphoreType.DMA,
        pltpu.SemaphoreType.DMA,
        pltpu.SemaphoreType.DMA,
        pltpu.SemaphoreType.DMA,
    ],
)

_BN = 4000


def _tc_body(f_ref, a_ref, coll_ref, srct_ref, xi_ref, o_ref):
    a = a_ref[0] + a_ref[1]
    f = f_ref[...]
    ssum = a[:, :_Q]
    wsum = a[:, _Q:_Q + 1]
    deg = jnp.maximum(a[:, _Q + 1:_Q + 2], 1.0)
    transport = (ssum - f * wsum) / deg * xi_ref[...]
    o_ref[...] = f - _DT * (transport - coll_ref[...] + srct_ref[...])


_tc_call = pl.pallas_call(
    _tc_body,
    out_shape=jax.ShapeDtypeStruct((_N, _Q), jnp.float32),
    grid=(_N // _BN,),
    in_specs=[
        pl.BlockSpec((_BN, _Q), lambda i: (i, 0)),
        pl.BlockSpec((_NC, _BN, _AW), lambda i: (0, i, 0)),
        pl.BlockSpec((_BN, _Q), lambda i: (i, 0)),
        pl.BlockSpec((_BN, _Q), lambda i: (i, 0)),
        pl.BlockSpec((1, _Q), lambda i: (0, 0)),
    ],
    out_specs=pl.BlockSpec((_BN, _Q), lambda i: (i, 0)),
)


def kernel(f_distribution, edge_index, edge_weight, collision_term,
           source_term, xi_velocities):
    src2 = edge_index.astype(jnp.int32)
    w1 = edge_weight
    fpad = jnp.concatenate(
        [f_distribution, jnp.ones((_N, 1), jnp.float32),
         jnp.zeros((_N, 15 - _Q), jnp.float32)], axis=1)
    zrows = jnp.zeros((_SLOT, _AW), jnp.float32)
    a = _sc_call(fpad, src2, w1, zrows)
    xi2 = xi_velocities.reshape(1, _Q)
    return _tc_call(f_distribution, a, collision_term, source_term, xi2)
